# baseline (speedup 1.0000x reference)
import functools

import jax
import jax.numpy as jnp
from jax.experimental import pallas as pl
from jax.experimental.pallas import tpu as pltpu

_NEG_SLOPE = 0.2


def _prep_kernel(x_ref, w_ref, as_ref, h_ref, a_s_ref):
    h = jnp.dot(x_ref[...], w_ref[...], preferred_element_type=jnp.float32)
    h_ref[...] = h
    a_s_ref[...] = jax.lax.dot_general(h, as_ref[...],
                                       (((1,), (1,)), ((), ())),
                                       preferred_element_type=jnp.float32)


def _gat_kernel(adj_ref, h_ref, a_s_ref, as_row_ref, ad_ref, b_ref, out_ref,
                p_ref, *, block_i, n_nodes):
    i = pl.program_id(0)

    h = h_ref[...]
    h_blk = h_ref[pl.ds(i * block_i, block_i), :]
    a_d = jax.lax.dot_general(ad_ref[...], h_blk,
                              (((1,), (1,)), ((), ())),
                              preferred_element_type=jnp.float32)

    s_mat = a_s_ref[...] + a_d
    s_mat = jnp.maximum(s_mat, _NEG_SLOPE * s_mat)
    s_mat = jnp.where(adj_ref[...] != 0, s_mat, -1e30)

    a_s_blk = jax.lax.dot_general(as_row_ref[...], h_blk,
                                  (((1,), (1,)), ((), ())),
                                  preferred_element_type=jnp.float32)
    diag = a_s_blk + a_d
    diag = jnp.maximum(diag, _NEG_SLOPE * diag)

    m = jnp.maximum(jnp.max(s_mat, axis=0, keepdims=True), diag)
    p_ref[...] = jnp.exp(s_mat - m)

    win = p_ref[pl.ds(i * block_i, block_i), :]
    r_idx = jax.lax.broadcasted_iota(jnp.int32, (block_i, block_i), 0)
    c_idx = jax.lax.broadcasted_iota(jnp.int32, (block_i, block_i), 1)
    p_diag = jnp.exp(diag - m)
    p_ref[pl.ds(i * block_i, block_i), :] = jnp.where(
        r_idx == c_idx, jnp.broadcast_to(p_diag, (block_i, block_i)), win)

    p = p_ref[...]
    denom = jnp.sum(p, axis=0, keepdims=True)
    inv = 1.0 / (denom + 1e-16)

    out_t = jax.lax.dot_general(h, p,
                                (((0,), (0,)), ((), ())),
                                preferred_element_type=jnp.float32)
    out_ref[...] = jnp.transpose(out_t * inv) + b_ref[...]


def kernel(x, adj, W, att_src, att_dst, bias):
    n, in_f = x.shape
    f = W.shape[1]
    att_s = att_src.reshape(1, f)
    att_d = att_dst.reshape(1, f)
    b = bias.reshape(1, f)

    h, a_s = pl.pallas_call(
        _prep_kernel,
        out_shape=[
            jax.ShapeDtypeStruct((n, f), jnp.float32),
            jax.ShapeDtypeStruct((n, 1), jnp.float32),
        ],
    )(x, W, att_s)

    block_i = 256
    grid = (n // block_i,)

    out = pl.pallas_call(
        functools.partial(_gat_kernel, block_i=block_i, n_nodes=n),
        grid=grid,
        in_specs=[
            pl.BlockSpec((n, block_i), lambda i: (0, i)),
            pl.BlockSpec((n, f), lambda i: (0, 0)),
            pl.BlockSpec((n, 1), lambda i: (0, 0)),
            pl.BlockSpec((1, f), lambda i: (0, 0)),
            pl.BlockSpec((1, f), lambda i: (0, 0)),
            pl.BlockSpec((1, f), lambda i: (0, 0)),
        ],
        out_specs=pl.BlockSpec((block_i, f), lambda i: (i, 0)),
        out_shape=jax.ShapeDtypeStruct((n, f), jnp.float32),
        scratch_shapes=[pltpu.VMEM((n, block_i), jnp.float32)],
        compiler_params=pltpu.CompilerParams(
            dimension_semantics=("parallel",),
        ),
    )(adj, h, a_s, att_s, att_d, b)
    return out


# speedup vs baseline: 1.0937x; 1.0937x over previous
import functools

import jax
import jax.numpy as jnp
from jax.experimental import pallas as pl
from jax.experimental.pallas import tpu as pltpu

_NEG_SLOPE = 0.2


def _prep_kernel(x_ref, w_ref, as_ref, h_ref, a_s_ref):
    h = jnp.dot(x_ref[...], w_ref[...], preferred_element_type=jnp.float32)
    h_ref[...] = h
    a_s_ref[...] = jax.lax.dot_general(h, as_ref[...],
                                       (((1,), (1,)), ((), ())),
                                       preferred_element_type=jnp.float32)


def _gat_kernel(adj_ref, h_ref, a_s_ref, as_row_ref, ad_ref, b_ref, out_ref,
                p_ref, *, block_i, n_nodes):
    i = pl.program_id(0)

    h = h_ref[...]
    h_blk = h_ref[pl.ds(i * block_i, block_i), :]
    a_d = jax.lax.dot_general(ad_ref[...], h_blk,
                              (((1,), (1,)), ((), ())),
                              preferred_element_type=jnp.float32)

    s_mat = a_s_ref[...] + a_d
    s_mat = jnp.maximum(s_mat, _NEG_SLOPE * s_mat)
    s_mat = jnp.where(adj_ref[...] != 0, s_mat, -1e30)

    a_s_blk = jax.lax.dot_general(as_row_ref[...], h_blk,
                                  (((1,), (1,)), ((), ())),
                                  preferred_element_type=jnp.float32)
    diag = a_s_blk + a_d
    diag = jnp.maximum(diag, _NEG_SLOPE * diag)

    m = jnp.maximum(jnp.max(s_mat, axis=0, keepdims=True), diag)
    p_ref[...] = jnp.exp(s_mat - m)

    win = p_ref[pl.ds(i * block_i, block_i), :]
    r_idx = jax.lax.broadcasted_iota(jnp.int32, (block_i, block_i), 0)
    c_idx = jax.lax.broadcasted_iota(jnp.int32, (block_i, block_i), 1)
    p_diag = jnp.exp(diag - m)
    p_ref[pl.ds(i * block_i, block_i), :] = jnp.where(
        r_idx == c_idx, jnp.broadcast_to(p_diag, (block_i, block_i)), win)

    p = p_ref[...]
    denom = jnp.sum(p, axis=0, keepdims=True)
    inv = 1.0 / (denom + 1e-16)

    out_t = jax.lax.dot_general(h, p,
                                (((0,), (0,)), ((), ())),
                                preferred_element_type=jnp.float32)
    out_ref[...] = jnp.transpose(out_t * inv) + b_ref[...]


def kernel(x, adj, W, att_src, att_dst, bias):
    n, in_f = x.shape
    f = W.shape[1]
    att_s = att_src.reshape(1, f)
    att_d = att_dst.reshape(1, f)
    b = bias.reshape(1, f)

    h, a_s = pl.pallas_call(
        _prep_kernel,
        out_shape=[
            jax.ShapeDtypeStruct((n, f), jnp.float32),
            jax.ShapeDtypeStruct((n, 1), jnp.float32),
        ],
    )(x, W, att_s)

    block_i = 512
    grid = (n // block_i,)

    out = pl.pallas_call(
        functools.partial(_gat_kernel, block_i=block_i, n_nodes=n),
        grid=grid,
        in_specs=[
            pl.BlockSpec((n, block_i), lambda i: (0, i)),
            pl.BlockSpec((n, f), lambda i: (0, 0)),
            pl.BlockSpec((n, 1), lambda i: (0, 0)),
            pl.BlockSpec((1, f), lambda i: (0, 0)),
            pl.BlockSpec((1, f), lambda i: (0, 0)),
            pl.BlockSpec((1, f), lambda i: (0, 0)),
        ],
        out_specs=pl.BlockSpec((block_i, f), lambda i: (i, 0)),
        out_shape=jax.ShapeDtypeStruct((n, f), jnp.float32),
        scratch_shapes=[pltpu.VMEM((n, block_i), jnp.float32)],
        compiler_params=pltpu.CompilerParams(
            dimension_semantics=("parallel",),
        ),
    )(adj, h, a_s, att_s, att_d, b)
    return out


# speedup vs baseline: 1.1624x; 1.0628x over previous
import functools

import jax
import jax.numpy as jnp
from jax.experimental import pallas as pl
from jax.experimental.pallas import tpu as pltpu

_NEG_SLOPE = 0.2


def _gat_kernel(x_ref, adj_ref, w_ref, as_ref, ad_ref, b_ref, out_ref,
                p_ref, h_ref, *, block_i, n_nodes):
    i = pl.program_id(0)

    h = jnp.dot(x_ref[...], w_ref[...],
                preferred_element_type=jnp.float32)
    h_ref[...] = h
    a_s_row = jax.lax.dot_general(as_ref[...], h,
                                  (((1,), (1,)), ((), ())),
                                  preferred_element_type=jnp.float32)
    a_s = jnp.transpose(a_s_row)
    h_blk = h_ref[pl.ds(i * block_i, block_i), :]
    a_d = jax.lax.dot_general(ad_ref[...], h_blk,
                              (((1,), (1,)), ((), ())),
                              preferred_element_type=jnp.float32)

    s_mat = a_s + a_d
    s_mat = jnp.maximum(s_mat, _NEG_SLOPE * s_mat)
    s_mat = jnp.where(adj_ref[...] != 0, s_mat, -1e30)

    a_s_blk = jax.lax.dot_general(as_ref[...], h_blk,
                                  (((1,), (1,)), ((), ())),
                                  preferred_element_type=jnp.float32)
    diag = a_s_blk + a_d
    diag = jnp.maximum(diag, _NEG_SLOPE * diag)

    m = jnp.maximum(jnp.max(s_mat, axis=0, keepdims=True), diag)
    p_ref[...] = jnp.exp(s_mat - m)

    win = p_ref[pl.ds(i * block_i, block_i), :]
    r_idx = jax.lax.broadcasted_iota(jnp.int32, (block_i, block_i), 0)
    c_idx = jax.lax.broadcasted_iota(jnp.int32, (block_i, block_i), 1)
    p_diag = jnp.exp(diag - m)
    p_ref[pl.ds(i * block_i, block_i), :] = jnp.where(
        r_idx == c_idx, jnp.broadcast_to(p_diag, (block_i, block_i)), win)

    p = p_ref[...]
    denom = jnp.sum(p, axis=0, keepdims=True)
    inv = 1.0 / (denom + 1e-16)

    out_t = jax.lax.dot_general(h, p,
                                (((0,), (0,)), ((), ())),
                                preferred_element_type=jnp.float32)
    out_ref[...] = jnp.transpose(out_t * inv) + b_ref[...]


def kernel(x, adj, W, att_src, att_dst, bias):
    n, in_f = x.shape
    f = W.shape[1]
    att_s = att_src.reshape(1, f)
    att_d = att_dst.reshape(1, f)
    b = bias.reshape(1, f)

    block_i = 512
    grid = (n // block_i,)

    out = pl.pallas_call(
        functools.partial(_gat_kernel, block_i=block_i, n_nodes=n),
        grid=grid,
        in_specs=[
            pl.BlockSpec((n, in_f), lambda i: (0, 0)),
            pl.BlockSpec((n, block_i), lambda i: (0, i)),
            pl.BlockSpec((in_f, f), lambda i: (0, 0)),
            pl.BlockSpec((1, f), lambda i: (0, 0)),
            pl.BlockSpec((1, f), lambda i: (0, 0)),
            pl.BlockSpec((1, f), lambda i: (0, 0)),
        ],
        out_specs=pl.BlockSpec((block_i, f), lambda i: (i, 0)),
        out_shape=jax.ShapeDtypeStruct((n, f), jnp.float32),
        scratch_shapes=[pltpu.VMEM((n, block_i), jnp.float32),
                        pltpu.VMEM((n, f), jnp.float32)],
        compiler_params=pltpu.CompilerParams(
            dimension_semantics=("parallel",),
        ),
    )(x, adj, W, att_s, att_d, b)
    return out


# speedup vs baseline: 1.1641x; 1.0014x over previous
import functools

import jax
import jax.numpy as jnp
from jax.experimental import pallas as pl
from jax.experimental.pallas import tpu as pltpu

_NEG_SLOPE = 0.2


def _gat_kernel(x_ref, adj_ref, w_ref, as_ref, ad_ref, b_ref, out_ref,
                p_ref, h_ref, *, block_i, n_nodes):
    i = pl.program_id(0)
    f = w_ref.shape[1]
    att_s = as_ref[...].reshape(1, f)
    att_d = ad_ref[...].reshape(1, f)
    b = b_ref[...].reshape(1, f)

    h = jnp.dot(x_ref[...], w_ref[...],
                preferred_element_type=jnp.float32)
    h_ref[...] = h
    a_s_row = jax.lax.dot_general(att_s, h,
                                  (((1,), (1,)), ((), ())),
                                  preferred_element_type=jnp.float32)
    a_s = jnp.transpose(a_s_row)
    h_blk = h_ref[pl.ds(i * block_i, block_i), :]
    a_d = jax.lax.dot_general(att_d, h_blk,
                              (((1,), (1,)), ((), ())),
                              preferred_element_type=jnp.float32)

    s_mat = a_s + a_d
    s_mat = jnp.maximum(s_mat, _NEG_SLOPE * s_mat)
    s_mat = jnp.where(adj_ref[...] != 0, s_mat, -1e30)

    a_s_blk = jax.lax.dot_general(att_s, h_blk,
                                  (((1,), (1,)), ((), ())),
                                  preferred_element_type=jnp.float32)
    diag = a_s_blk + a_d
    diag = jnp.maximum(diag, _NEG_SLOPE * diag)

    m = jnp.maximum(jnp.max(s_mat, axis=0, keepdims=True), diag)
    p_ref[...] = jnp.exp(s_mat - m)

    win = p_ref[pl.ds(i * block_i, block_i), :]
    r_idx = jax.lax.broadcasted_iota(jnp.int32, (block_i, block_i), 0)
    c_idx = jax.lax.broadcasted_iota(jnp.int32, (block_i, block_i), 1)
    p_diag = jnp.exp(diag - m)
    p_ref[pl.ds(i * block_i, block_i), :] = jnp.where(
        r_idx == c_idx, jnp.broadcast_to(p_diag, (block_i, block_i)), win)

    p = p_ref[...]
    denom = jnp.sum(p, axis=0, keepdims=True)
    inv = 1.0 / (denom + 1e-16)

    out_t = jax.lax.dot_general(h, p,
                                (((0,), (0,)), ((), ())),
                                preferred_element_type=jnp.float32)
    out_ref[...] = jnp.transpose(out_t * inv) + b


def kernel(x, adj, W, att_src, att_dst, bias):
    n, in_f = x.shape
    f = W.shape[1]

    block_i = 512
    grid = (n // block_i,)

    out = pl.pallas_call(
        functools.partial(_gat_kernel, block_i=block_i, n_nodes=n),
        grid=grid,
        in_specs=[
            pl.BlockSpec((n, in_f), lambda i: (0, 0)),
            pl.BlockSpec((n, block_i), lambda i: (0, i)),
            pl.BlockSpec((in_f, f), lambda i: (0, 0)),
            pl.BlockSpec((1, 1, f), lambda i: (0, 0, 0)),
            pl.BlockSpec((1, 1, f), lambda i: (0, 0, 0)),
            pl.BlockSpec((f,), lambda i: (0,)),
        ],
        out_specs=pl.BlockSpec((block_i, f), lambda i: (i, 0)),
        out_shape=jax.ShapeDtypeStruct((n, f), jnp.float32),
        scratch_shapes=[pltpu.VMEM((n, block_i), jnp.float32),
                        pltpu.VMEM((n, f), jnp.float32)],
        compiler_params=pltpu.CompilerParams(
            dimension_semantics=("parallel",),
        ),
    )(x, adj, W, att_src, att_dst, bias)
    return out


# speedup vs baseline: 1.5326x; 1.3166x over previous
import functools

import jax
import jax.numpy as jnp
from jax.experimental import pallas as pl
from jax.experimental.pallas import tpu as pltpu

_NEG_SLOPE = 0.2


def _gat_kernel(x_ref, adj_ref, wt_ref, as_ref, ad_ref, b_ref, out_ref,
                p_ref, h_ref, *, block_i, n_nodes):
    i = pl.program_id(0)
    f = wt_ref.shape[0]
    att_s = as_ref[...].reshape(1, f)
    att_d = ad_ref[...].reshape(1, f)
    b_col = b_ref[...].reshape(f, 1)

    h = jax.lax.dot_general(x_ref[...], wt_ref[...],
                            (((1,), (1,)), ((), ())),
                            preferred_element_type=jnp.float32)
    h_ref[...] = h
    a_s_row = jax.lax.dot_general(att_s, h,
                                  (((1,), (1,)), ((), ())),
                                  preferred_element_type=jnp.float32)
    a_s = jnp.transpose(a_s_row)
    h_blk = h_ref[pl.ds(i * block_i, block_i), :]
    a_d = jax.lax.dot_general(att_d, h_blk,
                              (((1,), (1,)), ((), ())),
                              preferred_element_type=jnp.float32)

    s_mat = a_s + a_d
    s_mat = jnp.maximum(s_mat, _NEG_SLOPE * s_mat)
    s_mat = jnp.where(adj_ref[...] != 0, s_mat, -1e30)

    a_s_blk = jax.lax.dot_general(att_s, h_blk,
                                  (((1,), (1,)), ((), ())),
                                  preferred_element_type=jnp.float32)
    diag = a_s_blk + a_d
    diag = jnp.maximum(diag, _NEG_SLOPE * diag)

    m = jnp.maximum(jnp.max(s_mat, axis=0, keepdims=True), diag)
    p_ref[...] = jnp.exp(s_mat - m)

    win = p_ref[pl.ds(i * block_i, block_i), :]
    r_idx = jax.lax.broadcasted_iota(jnp.int32, (block_i, block_i), 0)
    c_idx = jax.lax.broadcasted_iota(jnp.int32, (block_i, block_i), 1)
    p_diag = jnp.exp(diag - m)
    p_ref[pl.ds(i * block_i, block_i), :] = jnp.where(
        r_idx == c_idx, jnp.broadcast_to(p_diag, (block_i, block_i)), win)

    p = p_ref[...]
    denom = jnp.sum(p, axis=0, keepdims=True)
    inv = 1.0 / (denom + 1e-16)

    out_t = jax.lax.dot_general(h, p,
                                (((0,), (0,)), ((), ())),
                                preferred_element_type=jnp.float32)
    out_ref[...] = out_t * inv + b_col


def kernel(x, adj, W, att_src, att_dst, bias):
    n, in_f = x.shape
    f = W.shape[1]

    block_i = 512
    grid = (n // block_i,)

    out_t = pl.pallas_call(
        functools.partial(_gat_kernel, block_i=block_i, n_nodes=n),
        grid=grid,
        in_specs=[
            pl.BlockSpec((n, in_f), lambda i: (0, 0)),
            pl.BlockSpec((n, block_i), lambda i: (0, i)),
            pl.BlockSpec((f, in_f), lambda i: (0, 0)),
            pl.BlockSpec((1, 1, f), lambda i: (0, 0, 0)),
            pl.BlockSpec((1, 1, f), lambda i: (0, 0, 0)),
            pl.BlockSpec((f,), lambda i: (0,)),
        ],
        out_specs=pl.BlockSpec((f, block_i), lambda i: (0, i)),
        out_shape=jax.ShapeDtypeStruct((f, n), jnp.float32),
        scratch_shapes=[pltpu.VMEM((n, block_i), jnp.float32),
                        pltpu.VMEM((n, f), jnp.float32)],
        compiler_params=pltpu.CompilerParams(
            dimension_semantics=("parallel",),
        ),
    )(x, adj, W.T, att_src, att_dst, bias)
    return out_t.T


# speedup vs baseline: 1.5355x; 1.0019x over previous
import functools

import jax
import jax.numpy as jnp
from jax.experimental import pallas as pl
from jax.experimental.pallas import tpu as pltpu

_NEG_SLOPE = 0.2


def _gat_kernel(x_ref, adj_ref, wt_ref, as_ref, ad_ref, b_ref, out_ref,
                p_ref, h_ref, *, block_i, n_nodes):
    i = pl.program_id(0)
    f = wt_ref.shape[0]
    att_s = as_ref[...].reshape(1, f)
    att_d = ad_ref[...].reshape(1, f)
    b_col = b_ref[...].reshape(f, 1)

    h = jax.lax.dot_general(x_ref[...], wt_ref[...],
                            (((1,), (1,)), ((), ())),
                            preferred_element_type=jnp.float32)
    h_ref[...] = h
    a_s_row = jax.lax.dot_general(att_s, h,
                                  (((1,), (1,)), ((), ())),
                                  preferred_element_type=jnp.float32)
    a_s = jnp.transpose(a_s_row)
    h_blk = h_ref[pl.ds(i * block_i, block_i), :]
    a_d = jax.lax.dot_general(att_d, h_blk,
                              (((1,), (1,)), ((), ())),
                              preferred_element_type=jnp.float32)

    s_mat = a_s + a_d
    s_mat = jnp.maximum(s_mat, _NEG_SLOPE * s_mat)
    s_mat = jnp.where(adj_ref[...] != 0, s_mat, -1e30)

    a_s_blk = jax.lax.dot_general(att_s, h_blk,
                                  (((1,), (1,)), ((), ())),
                                  preferred_element_type=jnp.float32)
    diag = a_s_blk + a_d
    diag = jnp.maximum(diag, _NEG_SLOPE * diag)

    m = jnp.maximum(jnp.max(s_mat, axis=0, keepdims=True), diag)
    p_ref[...] = jnp.exp(s_mat - m)

    win = p_ref[pl.ds(i * block_i, block_i), :]
    r_idx = jax.lax.broadcasted_iota(jnp.int32, (block_i, block_i), 0)
    c_idx = jax.lax.broadcasted_iota(jnp.int32, (block_i, block_i), 1)
    p_diag = jnp.exp(diag - m)
    p_ref[pl.ds(i * block_i, block_i), :] = jnp.where(
        r_idx == c_idx, jnp.broadcast_to(p_diag, (block_i, block_i)), win)

    p = p_ref[...]
    denom = jnp.sum(p, axis=0, keepdims=True)
    inv = 1.0 / (denom + 1e-16)

    out_t = jax.lax.dot_general(h, p,
                                (((0,), (0,)), ((), ())),
                                preferred_element_type=jnp.float32)
    out_ref[...] = out_t * inv + b_col


def kernel(x, adj, W, att_src, att_dst, bias):
    n, in_f = x.shape
    f = W.shape[1]

    block_i = 512
    grid = (n // block_i,)

    out_t = pl.pallas_call(
        functools.partial(_gat_kernel, block_i=block_i, n_nodes=n),
        grid=grid,
        in_specs=[
            pl.BlockSpec((n, in_f), lambda i: (0, 0)),
            pl.BlockSpec((n, block_i), lambda i: (0, i)),
            pl.BlockSpec((f, in_f), lambda i: (0, 0)),
            pl.BlockSpec((1, 1, f), lambda i: (0, 0, 0)),
            pl.BlockSpec((1, 1, f), lambda i: (0, 0, 0)),
            pl.BlockSpec((f,), lambda i: (0,)),
        ],
        out_specs=pl.BlockSpec((f, block_i), lambda i: (0, i)),
        out_shape=jax.ShapeDtypeStruct((f, n), jnp.float32),
        scratch_shapes=[pltpu.VMEM((n, block_i), jnp.float32),
                        pltpu.VMEM((n, f), jnp.float32)],
        compiler_params=pltpu.CompilerParams(
            dimension_semantics=("arbitrary",),
        ),
    )(x, adj, W.T, att_src, att_dst, bias)
    return out_t.T


# speedup vs baseline: 1.5470x; 1.0075x over previous
import functools

import jax
import jax.numpy as jnp
from jax.experimental import pallas as pl
from jax.experimental.pallas import tpu as pltpu

_NEG_SLOPE = 0.2


def _gat_kernel(x_ref, adj_ref, wt_ref, as_ref, ad_ref, b_ref, out_ref,
                p_ref, h_ref, *, block_i, n_nodes):
    i = pl.program_id(0)
    f = wt_ref.shape[0]
    att_s = as_ref[...].reshape(1, f)
    att_d = ad_ref[...].reshape(1, f)
    b_col = b_ref[...].reshape(f, 1)

    h = jax.lax.dot_general(x_ref[...], wt_ref[...],
                            (((1,), (1,)), ((), ())),
                            preferred_element_type=jnp.float32)
    h_ref[...] = h
    a_s_row = jax.lax.dot_general(att_s, h,
                                  (((1,), (1,)), ((), ())),
                                  preferred_element_type=jnp.float32)
    a_s = jnp.transpose(a_s_row)
    h_blk = h_ref[pl.ds(i * block_i, block_i), :]
    a_d = jax.lax.dot_general(att_d, h_blk,
                              (((1,), (1,)), ((), ())),
                              preferred_element_type=jnp.float32)

    s_mat = a_s + a_d
    s_mat = jnp.maximum(s_mat, _NEG_SLOPE * s_mat)
    s_mat = jnp.where(adj_ref[...] != 0, s_mat, -1e30)

    a_s_blk = jax.lax.dot_general(att_s, h_blk,
                                  (((1,), (1,)), ((), ())),
                                  preferred_element_type=jnp.float32)
    diag = a_s_blk + a_d
    diag = jnp.maximum(diag, _NEG_SLOPE * diag)

    m = jnp.maximum(jnp.max(s_mat, axis=0, keepdims=True), diag)
    p_ref[...] = jnp.exp(s_mat - m)

    win = p_ref[pl.ds(i * block_i, block_i), :]
    r_idx = jax.lax.broadcasted_iota(jnp.int32, (block_i, block_i), 0)
    c_idx = jax.lax.broadcasted_iota(jnp.int32, (block_i, block_i), 1)
    p_diag = jnp.exp(diag - m)
    p_ref[pl.ds(i * block_i, block_i), :] = jnp.where(
        r_idx == c_idx, jnp.broadcast_to(p_diag, (block_i, block_i)), win)

    p = p_ref[...]
    denom = jnp.sum(p, axis=0, keepdims=True)
    inv = 1.0 / (denom + 1e-16)

    out_t = jax.lax.dot_general(h, p,
                                (((0,), (0,)), ((), ())),
                                preferred_element_type=jnp.float32)
    out_ref[...] = out_t * inv + b_col


def kernel(x, adj, W, att_src, att_dst, bias):
    n, in_f = x.shape
    f = W.shape[1]

    block_i = 1024
    grid = (n // block_i,)

    out_t = pl.pallas_call(
        functools.partial(_gat_kernel, block_i=block_i, n_nodes=n),
        grid=grid,
        in_specs=[
            pl.BlockSpec((n, in_f), lambda i: (0, 0)),
            pl.BlockSpec((n, block_i), lambda i: (0, i)),
            pl.BlockSpec((f, in_f), lambda i: (0, 0)),
            pl.BlockSpec((1, 1, f), lambda i: (0, 0, 0)),
            pl.BlockSpec((1, 1, f), lambda i: (0, 0, 0)),
            pl.BlockSpec((f,), lambda i: (0,)),
        ],
        out_specs=pl.BlockSpec((f, block_i), lambda i: (0, i)),
        out_shape=jax.ShapeDtypeStruct((f, n), jnp.float32),
        scratch_shapes=[pltpu.VMEM((n, block_i), jnp.float32),
                        pltpu.VMEM((n, f), jnp.float32)],
        compiler_params=pltpu.CompilerParams(
            dimension_semantics=("arbitrary",),
        ),
    )(x, adj, W.T, att_src, att_dst, bias)
    return out_t.T


# speedup vs baseline: 1.8082x; 1.1689x over previous
import functools

import jax
import jax.numpy as jnp
from jax.experimental import pallas as pl
from jax.experimental.pallas import tpu as pltpu

_NEG_SLOPE = 0.2


def _gat_kernel(x_ref, adj_ref, wt_ref, as_ref, ad_ref, b_ref, out_ref,
                h_ref, ht_ref, asc_ref, asr_ref, adr_ref, d_ref, sacc_ref,
                *, block_j, n_nodes):
    j = pl.program_id(0)
    nsteps = pl.num_programs(0)
    f = wt_ref.shape[0]

    @pl.when(j == 0)
    def _prologue():
        h = jax.lax.dot_general(x_ref[...], wt_ref[...],
                                (((1,), (1,)), ((), ())),
                                preferred_element_type=jnp.float32)
        h_ref[...] = h
        ht_ref[...] = jnp.transpose(h)
        a_s_row = jax.lax.dot_general(as_ref[...].reshape(1, f), h,
                                      (((1,), (1,)), ((), ())),
                                      preferred_element_type=jnp.float32)
        asr_ref[...] = a_s_row
        asc_ref[...] = jnp.transpose(a_s_row)
        adr_ref[...] = jax.lax.dot_general(ad_ref[...].reshape(1, f), h,
                                           (((1,), (1,)), ((), ())),
                                           preferred_element_type=jnp.float32)
        d_ref[...] = (
            jax.lax.broadcasted_iota(jnp.int32, (block_j, n_nodes), 1)
            - jax.lax.broadcasted_iota(jnp.int32, (block_j, n_nodes), 0))
        sacc_ref[...] = jnp.zeros((1, n_nodes), jnp.float32)
        out_ref[...] = jnp.zeros((f, n_nodes), jnp.float32)

    a_s_blk = asc_ref[pl.ds(j * block_j, block_j), :]
    e = a_s_blk + adr_ref[...]
    e = jnp.maximum(e, _NEG_SLOPE * e)
    keep = (adj_ref[...] != 0) & (d_ref[...] != j * block_j)
    p = jnp.where(keep, jnp.exp(e), 0.0)

    sacc_ref[...] += jnp.sum(p, axis=0, keepdims=True)
    h_blk = h_ref[pl.ds(j * block_j, block_j), :]
    out_ref[...] += jax.lax.dot_general(h_blk, p,
                                        (((0,), (0,)), ((), ())),
                                        preferred_element_type=jnp.float32)

    @pl.when(j == nsteps - 1)
    def _epilogue():
        diag = asr_ref[...] + adr_ref[...]
        diag = jnp.maximum(diag, _NEG_SLOPE * diag)
        p_diag = jnp.exp(diag)
        denom = sacc_ref[...] + p_diag
        inv = 1.0 / (denom + 1e-16)
        b_col = b_ref[...].reshape(f, 1)
        out_ref[...] = (out_ref[...] + ht_ref[...] * p_diag) * inv + b_col


def kernel(x, adj, W, att_src, att_dst, bias):
    n, in_f = x.shape
    f = W.shape[1]

    block_j = 512
    grid = (n // block_j,)

    out_t = pl.pallas_call(
        functools.partial(_gat_kernel, block_j=block_j, n_nodes=n),
        grid=grid,
        in_specs=[
            pl.BlockSpec((n, in_f), lambda j: (0, 0)),
            pl.BlockSpec((block_j, n), lambda j: (j, 0)),
            pl.BlockSpec((f, in_f), lambda j: (0, 0)),
            pl.BlockSpec((1, 1, f), lambda j: (0, 0, 0)),
            pl.BlockSpec((1, 1, f), lambda j: (0, 0, 0)),
            pl.BlockSpec((f,), lambda j: (0,)),
        ],
        out_specs=pl.BlockSpec((f, n), lambda j: (0, 0)),
        out_shape=jax.ShapeDtypeStruct((f, n), jnp.float32),
        scratch_shapes=[
            pltpu.VMEM((n, f), jnp.float32),
            pltpu.VMEM((f, n), jnp.float32),
            pltpu.VMEM((n, 1), jnp.float32),
            pltpu.VMEM((1, n), jnp.float32),
            pltpu.VMEM((1, n), jnp.float32),
            pltpu.VMEM((block_j, n), jnp.int32),
            pltpu.VMEM((1, n), jnp.float32),
        ],
        compiler_params=pltpu.CompilerParams(
            dimension_semantics=("arbitrary",),
        ),
    )(x, adj, W.T, att_src, att_dst, bias)
    return out_t.T


# speedup vs baseline: 1.9212x; 1.0624x over previous
import functools

import jax
import jax.numpy as jnp
from jax.experimental import pallas as pl
from jax.experimental.pallas import tpu as pltpu

_NEG_SLOPE = 0.2


def _gat_kernel(x_ref, adj_ref, wt_ref, as_ref, ad_ref, b_ref, out_ref,
                ha_ref, ht_ref, asc_ref, asr_ref, adr_ref, d_ref, acc_ref,
                *, block_j, n_nodes):
    j = pl.program_id(0)
    nsteps = pl.num_programs(0)
    f = wt_ref.shape[0]

    @pl.when(j == 0)
    def _prologue():
        h = jax.lax.dot_general(x_ref[...], wt_ref[...],
                                (((1,), (1,)), ((), ())),
                                preferred_element_type=jnp.float32)
        ha_ref[:, :f] = h
        ha_ref[:, f:] = jnp.ones((n_nodes, 1), jnp.float32)
        ht_ref[...] = jnp.transpose(h)
        a_s_row = jax.lax.dot_general(as_ref[...].reshape(1, f), h,
                                      (((1,), (1,)), ((), ())),
                                      preferred_element_type=jnp.float32)
        asr_ref[...] = a_s_row
        asc_ref[...] = jnp.transpose(a_s_row)
        adr_ref[...] = jax.lax.dot_general(ad_ref[...].reshape(1, f), h,
                                           (((1,), (1,)), ((), ())),
                                           preferred_element_type=jnp.float32)
        d_ref[...] = (
            jax.lax.broadcasted_iota(jnp.int32, (block_j, n_nodes), 1)
            - jax.lax.broadcasted_iota(jnp.int32, (block_j, n_nodes), 0))
        acc_ref[...] = jnp.zeros((f + 1, n_nodes), jnp.float32)

    a_s_blk = asc_ref[pl.ds(j * block_j, block_j), :]
    e = a_s_blk + adr_ref[...]
    e = jnp.maximum(e, _NEG_SLOPE * e)
    keep = (adj_ref[...] != 0) & (d_ref[...] != j * block_j)
    p = jnp.where(keep, jnp.exp(e), 0.0)

    ha_blk = ha_ref[pl.ds(j * block_j, block_j), :]
    acc_ref[...] += jax.lax.dot_general(ha_blk, p,
                                        (((0,), (0,)), ((), ())),
                                        preferred_element_type=jnp.float32)

    @pl.when(j == nsteps - 1)
    def _epilogue():
        diag = asr_ref[...] + adr_ref[...]
        diag = jnp.maximum(diag, _NEG_SLOPE * diag)
        p_diag = jnp.exp(diag)
        denom = acc_ref[f:, :] + p_diag
        inv = 1.0 / (denom + 1e-16)
        b_col = b_ref[...].reshape(f, 1)
        out_ref[...] = (acc_ref[:f, :] + ht_ref[...] * p_diag) * inv + b_col


def kernel(x, adj, W, att_src, att_dst, bias):
    n, in_f = x.shape
    f = W.shape[1]

    block_j = 512
    grid = (n // block_j,)

    out_t = pl.pallas_call(
        functools.partial(_gat_kernel, block_j=block_j, n_nodes=n),
        grid=grid,
        in_specs=[
            pl.BlockSpec((n, in_f), lambda j: (0, 0)),
            pl.BlockSpec((block_j, n), lambda j: (j, 0)),
            pl.BlockSpec((f, in_f), lambda j: (0, 0)),
            pl.BlockSpec((1, 1, f), lambda j: (0, 0, 0)),
            pl.BlockSpec((1, 1, f), lambda j: (0, 0, 0)),
            pl.BlockSpec((f,), lambda j: (0,)),
        ],
        out_specs=pl.BlockSpec((f, n), lambda j: (0, 0)),
        out_shape=jax.ShapeDtypeStruct((f, n), jnp.float32),
        scratch_shapes=[
            pltpu.VMEM((n, f + 1), jnp.float32),
            pltpu.VMEM((f, n), jnp.float32),
            pltpu.VMEM((n, 1), jnp.float32),
            pltpu.VMEM((1, n), jnp.float32),
            pltpu.VMEM((1, n), jnp.float32),
            pltpu.VMEM((block_j, n), jnp.int32),
            pltpu.VMEM((f + 1, n), jnp.float32),
        ],
        compiler_params=pltpu.CompilerParams(
            dimension_semantics=("arbitrary",),
        ),
    )(x, adj, W.T, att_src, att_dst, bias)
    return out_t.T
